# Initial kernel scaffold; baseline (speedup 1.0000x reference)
import jax
import jax.numpy as jnp
from jax.experimental import pallas as pl


def kernel(h, edge_index, W, a):
    raise NotImplementedError("write your pallas kernel here")


# speedup vs baseline: 9.9083x; 9.9083x over previous
import functools

import jax
import jax.numpy as jnp
from jax import lax
from jax.experimental import pallas as pl
from jax.experimental.pallas import tpu as pltpu
from jax.experimental.pallas import tpu_sc as plsc

N = 10000
D = 128
DX = 144
DENCOL = D
S1COL = D + 1
ALPHA = 0.2

NC = 2
NS = 16
NW = NC * NS
CHUNK = 64
E_IN = 320000
KCH = (E_IN + NW * CHUNK - 1) // (NW * CHUNK) + 1
E_PAD = NW * KCH * CHUNK
N_PAD = 10112
RPT = N_PAD // NS


def _prep_body(h_ref, w_ref, at_ref, whx_ref, s2_ref):
    wh = jnp.dot(h_ref[...], w_ref[...], preferred_element_type=jnp.float32)
    whx_ref[:, :D] = wh
    s1 = jnp.sum(wh * at_ref[0:1, :], axis=1)
    lane = lax.broadcasted_iota(jnp.int32, (h_ref.shape[0], 16), 1)
    whx_ref[:, D:DX] = jnp.where(
        lane == 0, 1.0, jnp.where(lane == 1, s1[:, None], 0.0)
    ).astype(jnp.float32)
    s2_ref[0, :] = jnp.sum(wh * at_ref[1:2, :], axis=1)


def _prep(h, W, aT):
    return pl.pallas_call(
        _prep_body,
        out_shape=[
            jax.ShapeDtypeStruct((N, DX), jnp.float32),
            jax.ShapeDtypeStruct((1, N), jnp.float32),
        ],
    )(h, W, aT)


def _sc_body(whx_hbm, srcs_hbm, dsts_hbm, s2_hbm, zeros_hbm,
             part0_hbm, part1_hbm,
             acc, s2_v, src_v, dst_v, rows_v, sem):
    cid = lax.axis_index("c")
    sid = lax.axis_index("s")
    wid = sid * NC + cid

    pltpu.sync_copy(s2_hbm.at[jnp.int32(0)], s2_v)

    r0 = sid * RPT
    pltpu.sync_copy(zeros_hbm.at[pl.ds(r0, RPT)], acc.at[pl.ds(r0, RPT)])
    plsc.subcore_barrier()

    iota16 = lax.iota(jnp.int32, 16)
    c0 = wid * KCH

    def chunk(ci, carry):
        pltpu.sync_copy(srcs_hbm.at[c0 + ci], src_v)
        pltpu.sync_copy(dsts_hbm.at[c0 + ci], dst_v)
        pltpu.async_copy(whx_hbm.at[src_v], rows_v, sem).wait()
        for j in range(CHUNK // 16):
            dv = dst_v[pl.ds(j * 16, 16)]
            s1g = plsc.load_gather(
                rows_v, [j * 16 + iota16, jnp.full((16,), S1COL, jnp.int32)])
            s2g = plsc.load_gather(s2_v, [dv])
            t = s1g + s2g
            ee = jnp.exp(jnp.maximum(t, ALPHA * t))
            for lane in range(16):
                esc = ee[lane]
                ei = j * 16 + lane
                for v in range(DX // 16):
                    sl = pl.ds(v * 16, 16)
                    rows_v[ei, sl] = rows_v[ei, sl] * esc
        pltpu.sync_copy(rows_v, acc.at[dst_v], add=True)
        return carry

    lax.fori_loop(jnp.int32(0), jnp.int32(KCH), chunk, jnp.int32(0))

    plsc.subcore_barrier()
    out_slice = pl.ds(r0, RPT)

    @pl.when(cid == 0)
    def _():
        pltpu.sync_copy(acc.at[out_slice], part0_hbm.at[out_slice])

    @pl.when(cid == 1)
    def _():
        pltpu.sync_copy(acc.at[out_slice], part1_hbm.at[out_slice])


def _sc_aggregate(whx, srcs, dsts, s2, zeros):
    mesh = plsc.VectorSubcoreMesh(core_axis_name="c", subcore_axis_name="s")
    kern = functools.partial(
        pl.kernel,
        mesh=mesh,
        compiler_params=pltpu.CompilerParams(
            needs_layout_passes=False, use_tc_tiling_on_sc=False),
        out_type=[
            jax.ShapeDtypeStruct((N_PAD, DX), jnp.float32),
            jax.ShapeDtypeStruct((N_PAD, DX), jnp.float32),
        ],
        scratch_types=[
            pltpu.VMEM_SHARED((N_PAD, DX), jnp.float32),
            pltpu.VMEM((N,), jnp.float32),
            pltpu.VMEM((CHUNK,), jnp.int32),
            pltpu.VMEM((CHUNK,), jnp.int32),
            pltpu.VMEM((CHUNK, DX), jnp.float32),
            pltpu.SemaphoreType.DMA,
        ],
    )(_sc_body)
    return kern(whx, srcs, dsts, s2, zeros)


def _final_body(p0_ref, p1_ref, h_ref, out_ref):
    num = p0_ref[:, :D] + p1_ref[:, :D]
    den = p0_ref[:, D:D + 1] + p1_ref[:, D:D + 1]
    out_ref[...] = h_ref[...] + jnp.where(den > 0, num / den, 0.0)


def _finalize(p0, p1, h):
    blk = 2000
    return pl.pallas_call(
        _final_body,
        grid=(N // blk,),
        in_specs=[
            pl.BlockSpec((blk, DX), lambda i: (i, 0)),
            pl.BlockSpec((blk, DX), lambda i: (i, 0)),
            pl.BlockSpec((blk, D), lambda i: (i, 0)),
        ],
        out_specs=pl.BlockSpec((blk, D), lambda i: (i, 0)),
        out_shape=jax.ShapeDtypeStruct((N, D), jnp.float32),
    )(p0, p1, h)


def kernel(h, edge_index, W, a):
    with jax.enable_x64(False):
        return _kernel32(h, edge_index, W, a)


def _kernel32(h, edge_index, W, a):
    h = h.astype(jnp.float32)
    src = edge_index[0].astype(jnp.int32)
    dst = edge_index[1].astype(jnp.int32)
    pad = E_PAD - src.shape[0]
    src = jnp.concatenate([src, jnp.zeros((pad,), jnp.int32)])
    dst = jnp.concatenate([dst, jnp.full((pad,), N, jnp.int32)])
    srcs = src.reshape(NW * KCH, CHUNK)
    dsts = dst.reshape(NW * KCH, CHUNK)
    aT = a.astype(jnp.float32).reshape(2, D)

    whx, s2 = _prep(h, W.astype(jnp.float32), aT)
    zeros = jnp.zeros((N_PAD, DX), jnp.float32)
    p0, p1 = _sc_aggregate(whx, srcs, dsts, s2, zeros)
    return _finalize(p0, p1, h)


# speedup vs baseline: 14.6641x; 1.4800x over previous
import functools

import jax
import jax.numpy as jnp
from jax import lax
from jax.experimental import pallas as pl
from jax.experimental.pallas import tpu as pltpu
from jax.experimental.pallas import tpu_sc as plsc

N = 10000
D = 128
DX = 144
DENCOL = D
S1COL = D + 1
ALPHA = 0.2

NC = 2
NS = 16
NW = NC * NS
CHUNK = 64
E_IN = 320000
KCH = (E_IN + NW * CHUNK - 1) // (NW * CHUNK) + 1
E_PAD = NW * KCH * CHUNK
N_PAD = 10112
RPT = N_PAD // NS


def _prep_body(h_ref, w_ref, at_ref, whx_ref, s2_ref):
    wh = jnp.dot(h_ref[...], w_ref[...], preferred_element_type=jnp.float32)
    whx_ref[:, :D] = wh
    s1 = jnp.sum(wh * at_ref[0:1, :], axis=1)
    lane = lax.broadcasted_iota(jnp.int32, (h_ref.shape[0], 16), 1)
    whx_ref[:, D:DX] = jnp.where(
        lane == 0, 1.0, jnp.where(lane == 1, s1[:, None], 0.0)
    ).astype(jnp.float32)
    s2_ref[0, :] = jnp.sum(wh * at_ref[1:2, :], axis=1)


def _prep(h, W, aT):
    return pl.pallas_call(
        _prep_body,
        out_shape=[
            jax.ShapeDtypeStruct((N, DX), jnp.float32),
            jax.ShapeDtypeStruct((1, N), jnp.float32),
        ],
    )(h, W, aT)


def _sc_body(whx_hbm, sd_hbm, s2_hbm, zeros_hbm,
             part0_hbm, part1_hbm,
             acc, s2_v, sd_v, rows_v,
             semg0, semg1, sems0, sems1, semi0, semi1):
    cid = lax.axis_index("c")
    sid = lax.axis_index("s")
    wid = sid * NC + cid

    semg = (semg0, semg1)
    sems = (sems0, sems1)
    semi = (semi0, semi1)

    pltpu.sync_copy(s2_hbm.at[jnp.int32(0)], s2_v)

    r0 = sid * RPT
    pltpu.sync_copy(zeros_hbm.at[pl.ds(r0, RPT)], acc.at[pl.ds(r0, RPT)])
    plsc.subcore_barrier()

    iota16 = lax.iota(jnp.int32, 16)
    c0 = wid * KCH

    def fire_idx(ci, b):
        pltpu.async_copy(sd_hbm.at[c0 + ci], sd_v.at[b], semi[b])

    def wait_idx(b):
        pltpu.make_async_copy(sd_hbm.at[c0], sd_v.at[b], semi[b]).wait()

    def fire_gather(b):
        pltpu.async_copy(whx_hbm.at[sd_v.at[b, jnp.int32(0)]],
                         rows_v.at[b], semg[b])

    def wait_gather(b):
        pltpu.make_async_copy(whx_hbm.at[sd_v.at[b, jnp.int32(0)]],
                              rows_v.at[b], semg[b]).wait()

    for b in range(2):
        fire_idx(jnp.int32(b), b)
    for b in range(2):
        wait_idx(b)
        fire_gather(b)

    def compute(b):
        for j in range(CHUNK // 16):
            dv = sd_v[b, 1, pl.ds(j * 16, 16)]
            s1g = plsc.load_gather(
                rows_v.at[b],
                [j * 16 + iota16, jnp.full((16,), S1COL, jnp.int32)])
            s2g = plsc.load_gather(s2_v, [dv])
            t = s1g + s2g
            ee = jnp.exp(jnp.maximum(t, ALPHA * t))
            for lane in range(16):
                esc = ee[lane]
                ei = j * 16 + lane
                for v in range(DX // 16):
                    sl = pl.ds(v * 16, 16)
                    rows_v[b, ei, sl] = rows_v[b, ei, sl] * esc

    def pair(p, carry):
        for b in range(2):
            ci = p * 2 + b
            wait_gather(b)
            compute(b)
            pltpu.async_copy(rows_v.at[b], acc.at[sd_v.at[b, jnp.int32(1)]],
                             sems[b], add=True).wait()
            ci2 = jnp.minimum(ci + 2, KCH - 1)
            fire_idx(ci2, b)
            wait_idx(b)
            fire_gather(b)
        return carry

    lax.fori_loop(jnp.int32(0), jnp.int32(KCH // 2), pair, jnp.int32(0))

    for b in range(2):
        wait_gather(b)

    plsc.subcore_barrier()
    out_slice = pl.ds(r0, RPT)

    @pl.when(cid == 0)
    def _():
        pltpu.sync_copy(acc.at[out_slice], part0_hbm.at[out_slice])

    @pl.when(cid == 1)
    def _():
        pltpu.sync_copy(acc.at[out_slice], part1_hbm.at[out_slice])


def _sc_aggregate(whx, sd, s2, zeros):
    mesh = plsc.VectorSubcoreMesh(core_axis_name="c", subcore_axis_name="s")
    kern = functools.partial(
        pl.kernel,
        mesh=mesh,
        compiler_params=pltpu.CompilerParams(
            needs_layout_passes=False, use_tc_tiling_on_sc=False),
        out_type=[
            jax.ShapeDtypeStruct((N_PAD, DX), jnp.float32),
            jax.ShapeDtypeStruct((N_PAD, DX), jnp.float32),
        ],
        scratch_types=[
            pltpu.VMEM_SHARED((N_PAD, DX), jnp.float32),
            pltpu.VMEM((N,), jnp.float32),
            pltpu.VMEM((2, 2, CHUNK), jnp.int32),
            pltpu.VMEM((2, CHUNK, DX), jnp.float32),
            pltpu.SemaphoreType.DMA,
            pltpu.SemaphoreType.DMA,
            pltpu.SemaphoreType.DMA,
            pltpu.SemaphoreType.DMA,
            pltpu.SemaphoreType.DMA,
            pltpu.SemaphoreType.DMA,
        ],
    )(_sc_body)
    return kern(whx, sd, s2, zeros)


def _final_body(p0_ref, p1_ref, h_ref, out_ref):
    num = p0_ref[:, :D] + p1_ref[:, :D]
    den = p0_ref[:, D:D + 1] + p1_ref[:, D:D + 1]
    out_ref[...] = h_ref[...] + jnp.where(den > 0, num / den, 0.0)


def _finalize(p0, p1, h):
    blk = 2000
    return pl.pallas_call(
        _final_body,
        grid=(N // blk,),
        in_specs=[
            pl.BlockSpec((blk, DX), lambda i: (i, 0)),
            pl.BlockSpec((blk, DX), lambda i: (i, 0)),
            pl.BlockSpec((blk, D), lambda i: (i, 0)),
        ],
        out_specs=pl.BlockSpec((blk, D), lambda i: (i, 0)),
        out_shape=jax.ShapeDtypeStruct((N, D), jnp.float32),
    )(p0, p1, h)


def kernel(h, edge_index, W, a):
    with jax.enable_x64(False):
        return _kernel32(h, edge_index, W, a)


def _kernel32(h, edge_index, W, a):
    h = h.astype(jnp.float32)
    src = edge_index[0].astype(jnp.int32)
    dst = edge_index[1].astype(jnp.int32)
    pad = E_PAD - src.shape[0]
    src = jnp.concatenate([src, jnp.zeros((pad,), jnp.int32)])
    dst = jnp.concatenate([dst, jnp.full((pad,), N, jnp.int32)])
    sd = jnp.stack([src.reshape(NW * KCH, CHUNK),
                    dst.reshape(NW * KCH, CHUNK)], axis=1)
    aT = a.astype(jnp.float32).reshape(2, D)

    whx, s2 = _prep(h, W.astype(jnp.float32), aT)
    zeros = jnp.zeros((N_PAD, DX), jnp.float32)
    p0, p1 = _sc_aggregate(whx, sd, s2, zeros)
    return _finalize(p0, p1, h)
